# baseline (speedup 1.0000x reference)
import functools
import jax
import jax.numpy as jnp
from jax import lax
from jax.experimental import pallas as pl
from jax.experimental.pallas import tpu as pltpu
from jax.experimental.pallas import tpu_sc as plsc

TC_BATCH = 3
CHUNK = 1024
W_SIZES = (1024, 1024, 2048, 4096)
PIECE = 64
NWORK = 32


def _tc_part(table, batch):
    n_rows, d_model = table.shape
    seq = n_rows
    n_chunks = seq // CHUNK
    w_offs = []
    off = 0
    for ln in W_SIZES:
        w_offs.append(off)
        off += ln
    assert off == seq

    def body(table_hbm, out_hbm, vmem, rsem, wsem):
        def read_copy(c):
            return pltpu.make_async_copy(
                table_hbm.at[pl.ds(c * CHUNK, CHUNK), :],
                vmem.at[pl.ds(c * CHUNK, CHUNK), :],
                rsem.at[c],
            )

        def write_copy(k, b):
            o, ln = w_offs[k], W_SIZES[k]
            return pltpu.make_async_copy(
                vmem.at[pl.ds(o, ln), :],
                out_hbm.at[b, pl.ds(o, ln), :],
                wsem.at[k, b],
            )

        for c in range(n_chunks):
            read_copy(c).start()

        chunks_waited = 0
        for k in range(len(W_SIZES)):
            need = (w_offs[k] + W_SIZES[k]) // CHUNK
            for c in range(chunks_waited, need):
                read_copy(c).wait()
            chunks_waited = need
            for b in range(batch):
                write_copy(k, b).start()

        for k in range(len(W_SIZES)):
            for b in range(batch):
                write_copy(k, b).wait()

    return pl.pallas_call(
        body,
        in_specs=[pl.BlockSpec(memory_space=pl.ANY)],
        out_specs=pl.BlockSpec(memory_space=pl.ANY),
        out_shape=jax.ShapeDtypeStruct((batch, seq, d_model), table.dtype),
        scratch_shapes=[
            pltpu.VMEM((seq, d_model), table.dtype),
            pltpu.SemaphoreType.DMA((n_chunks,)),
            pltpu.SemaphoreType.DMA((len(W_SIZES), batch)),
        ],
    )(table)


def _sc_part(table):
    n_rows, d_model = table.shape
    seq = n_rows
    chunk = seq // NWORK
    n_pieces = chunk // PIECE

    mesh = plsc.VectorSubcoreMesh(core_axis_name="c", subcore_axis_name="s")

    @functools.partial(
        pl.kernel,
        mesh=mesh,
        out_type=jax.ShapeDtypeStruct((seq, d_model), table.dtype),
        scratch_types=[
            pltpu.VMEM((2, PIECE, d_model), table.dtype),
            pltpu.SemaphoreType.DMA((2,)),
            pltpu.SemaphoreType.DMA((2,)),
        ],
    )
    def sc_copy(table_hbm, out_hbm, buf, rsem, wsem):
        c = lax.axis_index("c")
        s = lax.axis_index("s")
        wid = s * 2 + c
        off = wid * chunk

        def rd(k):
            return pltpu.make_async_copy(
                table_hbm.at[pl.ds(off + k * PIECE, PIECE), :],
                buf.at[k % 2],
                rsem.at[k % 2],
            )

        def wr(k):
            return pltpu.make_async_copy(
                buf.at[k % 2],
                out_hbm.at[pl.ds(off + k * PIECE, PIECE), :],
                wsem.at[k % 2],
            )

        rd(0).start()
        for k in range(n_pieces):
            if k + 1 < n_pieces:
                if k >= 1:
                    wr(k - 1).wait()
                rd(k + 1).start()
            rd(k).wait()
            wr(k).start()
        wr(n_pieces - 2).wait()
        wr(n_pieces - 1).wait()

    return sc_copy(table)


def kernel(inputs, table):
    batch, seq = inputs.shape
    sc_row = _sc_part(table)
    tc_rows = _tc_part(table, TC_BATCH)
    return jnp.concatenate([tc_rows, sc_row[None]], axis=0)


# speedup vs baseline: 3.1135x; 3.1135x over previous
import jax
import jax.numpy as jnp
from jax.experimental import pallas as pl
from jax.experimental.pallas import tpu as pltpu

CHUNK = 1024
W_SIZES = (1024, 7168)


def kernel(inputs, table):
    batch, seq = inputs.shape
    n_rows, d_model = table.shape
    n_chunks = seq // CHUNK
    w_offs = []
    off = 0
    for ln in W_SIZES:
        w_offs.append(off)
        off += ln
    assert off == seq

    def body(table_hbm, out_hbm, vmem, rsem, wsem):
        def read_copy(c):
            return pltpu.make_async_copy(
                table_hbm.at[pl.ds(c * CHUNK, CHUNK), :],
                vmem.at[pl.ds(c * CHUNK, CHUNK), :],
                rsem.at[c],
            )

        def write_copy(k, b):
            o, ln = w_offs[k], W_SIZES[k]
            return pltpu.make_async_copy(
                vmem.at[pl.ds(o, ln), :],
                out_hbm.at[b, pl.ds(o, ln), :],
                wsem.at[k, b],
            )

        for c in range(n_chunks):
            read_copy(c).start()

        chunks_waited = 0
        for k in range(len(W_SIZES)):
            need = (w_offs[k] + W_SIZES[k]) // CHUNK
            for c in range(chunks_waited, need):
                read_copy(c).wait()
            chunks_waited = need
            for b in range(batch):
                write_copy(k, b).start()

        for k in range(len(W_SIZES)):
            for b in range(batch):
                write_copy(k, b).wait()

    return pl.pallas_call(
        body,
        in_specs=[pl.BlockSpec(memory_space=pl.ANY)],
        out_specs=pl.BlockSpec(memory_space=pl.ANY),
        out_shape=jax.ShapeDtypeStruct((batch, seq, d_model), table.dtype),
        scratch_shapes=[
            pltpu.VMEM((seq, d_model), table.dtype),
            pltpu.SemaphoreType.DMA((n_chunks,)),
            pltpu.SemaphoreType.DMA((len(W_SIZES), batch)),
        ],
    )(table)


# speedup vs baseline: 3.3934x; 1.0899x over previous
import jax
import jax.numpy as jnp
from jax.experimental import pallas as pl
from jax.experimental.pallas import tpu as pltpu

CHUNK = 1024
W_SIZES = (2048, 2048, 4096)


def kernel(inputs, table):
    batch, seq = inputs.shape
    n_rows, d_model = table.shape
    n_chunks = seq // CHUNK
    w_offs = []
    off = 0
    for ln in W_SIZES:
        w_offs.append(off)
        off += ln
    assert off == seq

    def body(table_hbm, out_hbm, vmem, rsem, wsem):
        def read_copy(c):
            return pltpu.make_async_copy(
                table_hbm.at[pl.ds(c * CHUNK, CHUNK), :],
                vmem.at[pl.ds(c * CHUNK, CHUNK), :],
                rsem.at[c],
            )

        def write_copy(k, b):
            o, ln = w_offs[k], W_SIZES[k]
            return pltpu.make_async_copy(
                vmem.at[pl.ds(o, ln), :],
                out_hbm.at[b, pl.ds(o, ln), :],
                wsem.at[k, b],
            )

        for c in range(n_chunks):
            read_copy(c).start()

        chunks_waited = 0
        for k in range(len(W_SIZES)):
            need = (w_offs[k] + W_SIZES[k]) // CHUNK
            for c in range(chunks_waited, need):
                read_copy(c).wait()
            chunks_waited = need
            for b in range(batch):
                write_copy(k, b).start()

        for k in range(len(W_SIZES)):
            for b in range(batch):
                write_copy(k, b).wait()

    return pl.pallas_call(
        body,
        in_specs=[pl.BlockSpec(memory_space=pl.ANY)],
        out_specs=pl.BlockSpec(memory_space=pl.ANY),
        out_shape=jax.ShapeDtypeStruct((batch, seq, d_model), table.dtype),
        scratch_shapes=[
            pltpu.VMEM((seq, d_model), table.dtype),
            pltpu.SemaphoreType.DMA((n_chunks,)),
            pltpu.SemaphoreType.DMA((len(W_SIZES), batch)),
        ],
    )(table)


# speedup vs baseline: 3.4287x; 1.0104x over previous
import jax
import jax.numpy as jnp
from jax.experimental import pallas as pl
from jax.experimental.pallas import tpu as pltpu

S_BLK = 4096
NBUF = 2


def kernel(inputs, table):
    batch, seq = inputs.shape
    n_rows, d_model = table.shape
    n_blocks = seq // S_BLK

    def body(table_hbm, out_hbm, vmem, in_sems, out_sems):
        i = pl.program_id(0)

        def in_copy(j, slot):
            return pltpu.make_async_copy(
                table_hbm.at[pl.ds(j * S_BLK, S_BLK), :],
                vmem.at[slot],
                in_sems.at[slot],
            )

        def out_copy(j, slot, b):
            return pltpu.make_async_copy(
                vmem.at[slot],
                out_hbm.at[b, pl.ds(j * S_BLK, S_BLK), :],
                out_sems.at[slot, b],
            )

        slot = jax.lax.rem(i, NBUF)

        @pl.when(i == 0)
        def _():
            in_copy(0, 0).start()

        @pl.when(i + 1 < n_blocks)
        def _():
            nslot = jax.lax.rem(i + 1, NBUF)

            @pl.when(i + 1 >= NBUF)
            def _():
                for b in range(batch):
                    out_copy(i + 1 - NBUF, nslot, b).wait()

            in_copy(i + 1, nslot).start()

        in_copy(i, slot).wait()
        for b in range(batch):
            out_copy(i, slot, b).start()

        @pl.when(i == n_blocks - 1)
        def _():
            for j in range(max(0, n_blocks - NBUF), n_blocks):
                for b in range(batch):
                    out_copy(j, j % NBUF, b).wait()

    return pl.pallas_call(
        body,
        grid=(n_blocks,),
        in_specs=[pl.BlockSpec(memory_space=pl.ANY)],
        out_specs=pl.BlockSpec(memory_space=pl.ANY),
        out_shape=jax.ShapeDtypeStruct((batch, seq, d_model), table.dtype),
        scratch_shapes=[
            pltpu.VMEM((NBUF, S_BLK, d_model), table.dtype),
            pltpu.SemaphoreType.DMA((NBUF,)),
            pltpu.SemaphoreType.DMA((NBUF, batch)),
        ],
    )(table)
